# baseline (speedup 1.0000x reference)
import functools

import jax
import jax.numpy as jnp
from jax import lax
from jax.experimental import pallas as pl
from jax.experimental.pallas import tpu as pltpu
from jax.experimental.pallas import tpu_sc as plsc

N = 10000
E = 320000
G = 64
F = 64
NODE_IN = 128
EDGE_IN = 16
LAYERS = 3
_C = float((1.0 + 1e-5) ** -0.5)

_NC = 2
_NS = 16
_NW = _NC * _NS
_CH = 80
_EPT = E // _NW
_NCHUNK = _EPT // _CH
_RB = 80
_NRB = N // _RB


def _embed_call(x, Wn, bn, A, cb):
    B = 2000

    def body(x_ref, wn_ref, bn_ref, a_ref, c_ref, h_ref, p_ref):
        h = jnp.dot(x_ref[...], wn_ref[...],
                    preferred_element_type=jnp.float32) + bn_ref[...]
        h_ref[...] = h
        p = jnp.dot(h, a_ref[...],
                    preferred_element_type=jnp.float32) + c_ref[...]
        p_ref[...] = jnp.concatenate([p, jnp.zeros_like(p)], axis=-1)

    return pl.pallas_call(
        body,
        grid=(N // B,),
        in_specs=[
            pl.BlockSpec((B, NODE_IN), lambda i: (i, 0)),
            pl.BlockSpec((NODE_IN, F), lambda i: (0, 0)),
            pl.BlockSpec((1, F), lambda i: (0, 0)),
            pl.BlockSpec((F, F), lambda i: (0, 0)),
            pl.BlockSpec((1, F), lambda i: (0, 0)),
        ],
        out_specs=[
            pl.BlockSpec((B, F), lambda i: (i, 0)),
            pl.BlockSpec((B, 2 * F), lambda i: (i, 0)),
        ],
        out_shape=[jax.ShapeDtypeStruct((N, F), jnp.float32),
                   jax.ShapeDtypeStruct((N, 2 * F), jnp.float32)],
    )(x, Wn, bn, A, cb)


def _q_call(edge_attr, Wc_all):
    B = 8000

    def body(ea_ref, w_ref, q0_ref, q1_ref, q2_ref):
        q = jnp.dot(ea_ref[...], w_ref[...], preferred_element_type=jnp.float32)
        q0_ref[...] = q[:, :F]
        q1_ref[...] = q[:, F:2 * F]
        q2_ref[...] = q[:, 2 * F:]

    return pl.pallas_call(
        body,
        grid=(E // B,),
        in_specs=[
            pl.BlockSpec((B, EDGE_IN), lambda i: (i, 0)),
            pl.BlockSpec((EDGE_IN, LAYERS * F), lambda i: (0, 0)),
        ],
        out_specs=[pl.BlockSpec((B, F), lambda i: (i, 0))] * LAYERS,
        out_shape=[jax.ShapeDtypeStruct((E, F), jnp.float32)] * LAYERS,
    )(edge_attr, Wc_all)


def _update_call(h, parts, Wt, Wb, bu, A=None, cb=None):
    B = 2000
    with_p = A is not None

    def body(h_ref, p0_ref, p1_ref, wt_ref, wb_ref, bu_ref, *rest):
        if with_p:
            a_ref, c_ref, hn_ref, pn_ref = rest
        else:
            (hn_ref,) = rest
        u = (jnp.dot(h_ref[...], wt_ref[...],
                     preferred_element_type=jnp.float32)
             + jnp.dot(p0_ref[0] + p1_ref[0], wb_ref[...],
                       preferred_element_type=jnp.float32)
             + bu_ref[...])
        hn = _C * jnp.maximum(u, 0.0)
        hn_ref[...] = hn
        if with_p:
            pn = jnp.dot(hn, a_ref[...],
                         preferred_element_type=jnp.float32) + c_ref[...]
            pn_ref[...] = jnp.concatenate([pn, jnp.zeros_like(pn)], axis=-1)

    in_specs = [
        pl.BlockSpec((B, F), lambda i: (i, 0)),
        pl.BlockSpec((1, B, F), lambda i: (0, i, 0)),
        pl.BlockSpec((1, B, F), lambda i: (1, i, 0)),
        pl.BlockSpec((F, F), lambda i: (0, 0)),
        pl.BlockSpec((F, F), lambda i: (0, 0)),
        pl.BlockSpec((1, F), lambda i: (0, 0)),
    ]
    args = [h, parts, parts, Wt, Wb, bu]
    n_out = 1
    out_specs = [pl.BlockSpec((B, F), lambda i: (i, 0))]
    out_shape = [jax.ShapeDtypeStruct((N, F), jnp.float32)]
    if with_p:
        in_specs += [pl.BlockSpec((F, F), lambda i: (0, 0)),
                     pl.BlockSpec((1, F), lambda i: (0, 0))]
        args += [A, cb]
        out_specs.append(pl.BlockSpec((B, 2 * F), lambda i: (i, 0)))
        out_shape.append(jax.ShapeDtypeStruct((N, 2 * F), jnp.float32))
    return pl.pallas_call(
        body,
        grid=(N // B,),
        in_specs=in_specs,
        out_specs=out_specs,
        out_shape=out_shape,
    )(*args)


def _pool_readout_call(h, batch3, W1, b1, W2, b2):
    B = 2000
    S = N // B

    def body(b_ref, h_ref, w1_ref, b1_ref, w2_ref, b2_ref, o_ref, acc):
        i = pl.program_id(0)

        @pl.when(i == 0)
        def _():
            acc[...] = jnp.zeros_like(acc)

        ids = b_ref[0, 0, :]
        onehot = (ids[:, None] == lax.broadcasted_iota(jnp.int32, (B, G), 1)
                  ).astype(jnp.float32)
        acc[...] += lax.dot_general(onehot, h_ref[...],
                                    (((0,), (0,)), ((), ())),
                                    preferred_element_type=jnp.float32)

        @pl.when(i == S - 1)
        def _():
            g = acc[...]
            r = jnp.maximum(jnp.dot(g, w1_ref[...],
                                    preferred_element_type=jnp.float32)
                            + b1_ref[...], 0.0)
            o_ref[...] = jnp.dot(r, w2_ref[...],
                                 preferred_element_type=jnp.float32) + b2_ref[...]

    return pl.pallas_call(
        body,
        grid=(S,),
        in_specs=[
            pl.BlockSpec((1, 1, B), lambda i: (i, 0, 0)),
            pl.BlockSpec((B, F), lambda i: (i, 0)),
            pl.BlockSpec((F, F), lambda i: (0, 0)),
            pl.BlockSpec((1, F), lambda i: (0, 0)),
            pl.BlockSpec((F, 1), lambda i: (0, 0)),
            pl.BlockSpec((1, 1), lambda i: (0, 0)),
        ],
        out_specs=pl.BlockSpec((G, 1), lambda i: (0, 0)),
        out_shape=jax.ShapeDtypeStruct((G, 1), jnp.float32),
        scratch_shapes=[pltpu.VMEM((G, G), jnp.float32)],
    )(batch3, h, W1, b1, W2, b2)


def _edge_pass(p, q, src, dst):
    mesh = plsc.VectorSubcoreMesh(core_axis_name="c", subcore_axis_name="s")

    @functools.partial(
        pl.kernel,
        out_type=jax.ShapeDtypeStruct((_NC, N, F), jnp.float32),
        mesh=mesh,
        scratch_types=[
            pltpu.VMEM((_CH,), jnp.int32),
            pltpu.VMEM((_CH,), jnp.int32),
            pltpu.VMEM((_CH, 2 * F), jnp.float32),
            pltpu.VMEM((_CH, F), jnp.float32),
            pltpu.VMEM((_RB, F), jnp.float32),
            pltpu.VMEM_SHARED((N, F), jnp.float32),
            pltpu.SemaphoreType.DMA,
        ],
    )
    def ker(p_hbm, q_hbm, src_hbm, dst_hbm, out_hbm,
            sidx, didx, pg, qv, zb, aggr, sem):
        ci = lax.axis_index("c")
        si = lax.axis_index("s")
        wid = ci * _NS + si

        @pl.loop(0, _RB)
        def _(i):
            for j in range(F // 16):
                zb[i, pl.ds(j * 16, 16)] = jnp.zeros((16,), jnp.float32)

        nblk = jnp.where(si < _NRB % _NS, _NRB // _NS + 1, _NRB // _NS)

        @pl.loop(0, nblk)
        def _(k):
            pltpu.sync_copy(zb, aggr.at[pl.ds((si + k * _NS) * _RB, _RB)])

        plsc.subcore_barrier()

        @pl.loop(0, _NCHUNK)
        def _(t):
            base = wid * _EPT + t * _CH
            pltpu.sync_copy(src_hbm.at[pl.ds(base, _CH)], sidx)
            pltpu.sync_copy(dst_hbm.at[pl.ds(base, _CH)], didx)
            dg = pltpu.async_copy(p_hbm.at[sidx], pg, sem)
            pltpu.sync_copy(q_hbm.at[pl.ds(base, _CH)], qv)
            dg.wait()

            @pl.loop(0, _CH, unroll=4)
            def _(i):
                for j in range(F // 16):
                    sl = pl.ds(j * 16, 16)
                    qv[i, sl] = jnp.maximum(pg[i, sl] + qv[i, sl], 0.0)

            pltpu.sync_copy(qv, aggr.at[didx], add=True)

        plsc.subcore_barrier()

        @pl.loop(0, nblk)
        def _(k):
            r0 = (si + k * _NS) * _RB
            pltpu.sync_copy(aggr.at[pl.ds(r0, _RB)],
                            out_hbm.at[ci].at[pl.ds(r0, _RB)])

    return ker(p, q, src, dst)


def kernel(x, edge_index, edge_attr, batch, params):
    Wn, bn = params["node_emb"]
    We, be = params["edge_emb"]
    layers = params["layers"]
    src = edge_index[0]
    dst = edge_index[1]

    Wm_top, Wm_bot, p_bias, Wc, Wt, Wb, bu = [], [], [], [], [], [], []
    for lp in layers:
        Wm = lp["Wm"]
        Wm_top.append(Wm[:F])
        Wm_bot.append(Wm[F:])
        p_bias.append((lp["bm"] + be @ Wm[F:]).reshape(1, F))
        Wc.append(We @ Wm[F:])
        Wu = lp["Wu"]
        Wt.append(Wu[:F])
        Wb.append(_C * Wu[F:])
        bu.append(lp["bu"].reshape(1, F))
    Wc_all = jnp.concatenate(Wc, axis=1)

    h, p = _embed_call(x, Wn, bn.reshape(1, F), Wm_top[0], p_bias[0])
    qs = _q_call(edge_attr, Wc_all)

    for l in range(LAYERS):
        parts = _edge_pass(p, qs[l], src, dst)
        if l + 1 < LAYERS:
            h, p = _update_call(h, parts, Wt[l], Wb[l], bu[l],
                                Wm_top[l + 1], p_bias[l + 1])
        else:
            (h,) = _update_call(h, parts, Wt[l], Wb[l], bu[l])

    W1, b1 = params["r1"]
    W2, b2 = params["r2"]
    batch3 = batch.reshape(N // 2000, 1, 2000)
    return _pool_readout_call(h, batch3, W1, b1.reshape(1, F),
                              W2, b2.reshape(1, 1))


# speedup vs baseline: 1.2941x; 1.2941x over previous
import functools

import jax
import jax.numpy as jnp
from jax import lax
from jax.experimental import pallas as pl
from jax.experimental.pallas import tpu as pltpu
from jax.experimental.pallas import tpu_sc as plsc

N = 10000
E = 320000
G = 64
F = 64
NODE_IN = 128
EDGE_IN = 16
LAYERS = 3
_C = float((1.0 + 1e-5) ** -0.5)

_NC = 2
_NS = 16
_NW = _NC * _NS
_CH = 80
_EPT = E // _NW
_NCHUNK = _EPT // _CH
_RB = 80
_NRB = N // _RB


def _embed_call(x, Wn, bn, A, cb):
    B = 2000

    def body(x_ref, wn_ref, bn_ref, a_ref, c_ref, h_ref, p_ref):
        h = jnp.dot(x_ref[...], wn_ref[...],
                    preferred_element_type=jnp.float32) + bn_ref[...]
        h_ref[...] = h
        p = jnp.dot(h, a_ref[...],
                    preferred_element_type=jnp.float32) + c_ref[...]
        p_ref[...] = jnp.concatenate([p, jnp.zeros_like(p)], axis=-1)

    return pl.pallas_call(
        body,
        grid=(N // B,),
        in_specs=[
            pl.BlockSpec((B, NODE_IN), lambda i: (i, 0)),
            pl.BlockSpec((NODE_IN, F), lambda i: (0, 0)),
            pl.BlockSpec((1, F), lambda i: (0, 0)),
            pl.BlockSpec((F, F), lambda i: (0, 0)),
            pl.BlockSpec((1, F), lambda i: (0, 0)),
        ],
        out_specs=[
            pl.BlockSpec((B, F), lambda i: (i, 0)),
            pl.BlockSpec((B, 2 * F), lambda i: (i, 0)),
        ],
        out_shape=[jax.ShapeDtypeStruct((N, F), jnp.float32),
                   jax.ShapeDtypeStruct((N, 2 * F), jnp.float32)],
    )(x, Wn, bn, A, cb)


def _q_call(edge_attr, Wc_all):
    B = 8000

    def body(ea_ref, w_ref, q0_ref, q1_ref, q2_ref):
        q = jnp.dot(ea_ref[...], w_ref[...], preferred_element_type=jnp.float32)
        q0_ref[...] = q[:, :F]
        q1_ref[...] = q[:, F:2 * F]
        q2_ref[...] = q[:, 2 * F:]

    return pl.pallas_call(
        body,
        grid=(E // B,),
        in_specs=[
            pl.BlockSpec((B, EDGE_IN), lambda i: (i, 0)),
            pl.BlockSpec((EDGE_IN, LAYERS * F), lambda i: (0, 0)),
        ],
        out_specs=[pl.BlockSpec((B, F), lambda i: (i, 0))] * LAYERS,
        out_shape=[jax.ShapeDtypeStruct((E, F), jnp.float32)] * LAYERS,
    )(edge_attr, Wc_all)


def _update_call(h, parts, Wt, Wb, bu, A=None, cb=None):
    B = 2000
    with_p = A is not None

    def body(h_ref, p0_ref, p1_ref, wt_ref, wb_ref, bu_ref, *rest):
        if with_p:
            a_ref, c_ref, hn_ref, pn_ref = rest
        else:
            (hn_ref,) = rest
        u = (jnp.dot(h_ref[...], wt_ref[...],
                     preferred_element_type=jnp.float32)
             + jnp.dot(p0_ref[0] + p1_ref[0], wb_ref[...],
                       preferred_element_type=jnp.float32)
             + bu_ref[...])
        hn = _C * jnp.maximum(u, 0.0)
        hn_ref[...] = hn
        if with_p:
            pn = jnp.dot(hn, a_ref[...],
                         preferred_element_type=jnp.float32) + c_ref[...]
            pn_ref[...] = jnp.concatenate([pn, jnp.zeros_like(pn)], axis=-1)

    in_specs = [
        pl.BlockSpec((B, F), lambda i: (i, 0)),
        pl.BlockSpec((1, B, F), lambda i: (0, i, 0)),
        pl.BlockSpec((1, B, F), lambda i: (1, i, 0)),
        pl.BlockSpec((F, F), lambda i: (0, 0)),
        pl.BlockSpec((F, F), lambda i: (0, 0)),
        pl.BlockSpec((1, F), lambda i: (0, 0)),
    ]
    args = [h, parts, parts, Wt, Wb, bu]
    n_out = 1
    out_specs = [pl.BlockSpec((B, F), lambda i: (i, 0))]
    out_shape = [jax.ShapeDtypeStruct((N, F), jnp.float32)]
    if with_p:
        in_specs += [pl.BlockSpec((F, F), lambda i: (0, 0)),
                     pl.BlockSpec((1, F), lambda i: (0, 0))]
        args += [A, cb]
        out_specs.append(pl.BlockSpec((B, 2 * F), lambda i: (i, 0)))
        out_shape.append(jax.ShapeDtypeStruct((N, 2 * F), jnp.float32))
    return pl.pallas_call(
        body,
        grid=(N // B,),
        in_specs=in_specs,
        out_specs=out_specs,
        out_shape=out_shape,
    )(*args)


def _pool_readout_call(h, batch3, W1, b1, W2, b2):
    B = 2000
    S = N // B

    def body(b_ref, h_ref, w1_ref, b1_ref, w2_ref, b2_ref, o_ref, acc):
        i = pl.program_id(0)

        @pl.when(i == 0)
        def _():
            acc[...] = jnp.zeros_like(acc)

        ids = b_ref[0, 0, :]
        onehot = (ids[:, None] == lax.broadcasted_iota(jnp.int32, (B, G), 1)
                  ).astype(jnp.float32)
        acc[...] += lax.dot_general(onehot, h_ref[...],
                                    (((0,), (0,)), ((), ())),
                                    preferred_element_type=jnp.float32)

        @pl.when(i == S - 1)
        def _():
            g = acc[...]
            r = jnp.maximum(jnp.dot(g, w1_ref[...],
                                    preferred_element_type=jnp.float32)
                            + b1_ref[...], 0.0)
            o_ref[...] = jnp.dot(r, w2_ref[...],
                                 preferred_element_type=jnp.float32) + b2_ref[...]

    return pl.pallas_call(
        body,
        grid=(S,),
        in_specs=[
            pl.BlockSpec((1, 1, B), lambda i: (i, 0, 0)),
            pl.BlockSpec((B, F), lambda i: (i, 0)),
            pl.BlockSpec((F, F), lambda i: (0, 0)),
            pl.BlockSpec((1, F), lambda i: (0, 0)),
            pl.BlockSpec((F, 1), lambda i: (0, 0)),
            pl.BlockSpec((1, 1), lambda i: (0, 0)),
        ],
        out_specs=pl.BlockSpec((G, 1), lambda i: (0, 0)),
        out_shape=jax.ShapeDtypeStruct((G, 1), jnp.float32),
        scratch_shapes=[pltpu.VMEM((G, G), jnp.float32)],
    )(batch3, h, W1, b1, W2, b2)


def _edge_pass(p, q, src, dst):
    mesh = plsc.VectorSubcoreMesh(core_axis_name="c", subcore_axis_name="s")

    @functools.partial(
        pl.kernel,
        out_type=jax.ShapeDtypeStruct((_NC, N, F), jnp.float32),
        mesh=mesh,
        scratch_types=[
            pltpu.VMEM((_CH,), jnp.int32),
            pltpu.VMEM((_CH,), jnp.int32),
            pltpu.VMEM((_CH, 2 * F), jnp.float32),
            pltpu.VMEM((_CH, F), jnp.float32),
            pltpu.VMEM((_RB, F), jnp.float32),
            pltpu.VMEM_SHARED((N, F), jnp.float32),
            pltpu.SemaphoreType.DMA,
        ],
    )
    def ker(p_hbm, q_hbm, src_hbm, dst_hbm, out_hbm,
            sidx, didx, pg, qv, zb, aggr, sem):
        ci = lax.axis_index("c")
        si = lax.axis_index("s")
        wid = ci * _NS + si

        @pl.loop(0, _RB)
        def _(i):
            for j in range(F // 16):
                zb[i, pl.ds(j * 16, 16)] = jnp.zeros((16,), jnp.float32)

        nblk = jnp.where(si < _NRB % _NS, _NRB // _NS + 1, _NRB // _NS)

        @pl.loop(0, nblk)
        def _(k):
            pltpu.sync_copy(zb, aggr.at[pl.ds((si + k * _NS) * _RB, _RB)])

        plsc.subcore_barrier()

        @pl.loop(0, _NCHUNK)
        def _(t):
            base = wid * _EPT + t * _CH
            pltpu.sync_copy(src_hbm.at[pl.ds(base, _CH)], sidx)
            pltpu.sync_copy(dst_hbm.at[pl.ds(base, _CH)], didx)
            dg = pltpu.async_copy(p_hbm.at[sidx], pg, sem)
            pltpu.sync_copy(q_hbm.at[pl.ds(base, _CH)], qv)
            dg.wait()

            @pl.loop(0, _CH)
            def _(i):
                for j in range(F // 16):
                    sl = pl.ds(j * 16, 16)
                    qv[i, sl] = jnp.maximum(pg[i, sl] + qv[i, sl], 0.0)

            pltpu.sync_copy(qv, aggr.at[didx], add=True)

        plsc.subcore_barrier()

        @pl.loop(0, nblk)
        def _(k):
            r0 = (si + k * _NS) * _RB
            pltpu.sync_copy(aggr.at[pl.ds(r0, _RB)],
                            out_hbm.at[ci].at[pl.ds(r0, _RB)])

    return ker(p, q, src, dst)


def kernel(x, edge_index, edge_attr, batch, params):
    Wn, bn = params["node_emb"]
    We, be = params["edge_emb"]
    layers = params["layers"]
    src = edge_index[0]
    dst = edge_index[1]

    Wm_top, Wm_bot, p_bias, Wc, Wt, Wb, bu = [], [], [], [], [], [], []
    for lp in layers:
        Wm = lp["Wm"]
        Wm_top.append(Wm[:F])
        Wm_bot.append(Wm[F:])
        p_bias.append((lp["bm"] + be @ Wm[F:]).reshape(1, F))
        Wc.append(We @ Wm[F:])
        Wu = lp["Wu"]
        Wt.append(Wu[:F])
        Wb.append(_C * Wu[F:])
        bu.append(lp["bu"].reshape(1, F))
    Wc_all = jnp.concatenate(Wc, axis=1)

    h, p = _embed_call(x, Wn, bn.reshape(1, F), Wm_top[0], p_bias[0])
    qs = _q_call(edge_attr, Wc_all)

    for l in range(LAYERS):
        parts = _edge_pass(p, qs[l], src, dst)
        if l + 1 < LAYERS:
            h, p = _update_call(h, parts, Wt[l], Wb[l], bu[l],
                                Wm_top[l + 1], p_bias[l + 1])
        else:
            (h,) = _update_call(h, parts, Wt[l], Wb[l], bu[l])

    W1, b1 = params["r1"]
    W2, b2 = params["r2"]
    batch3 = batch.reshape(N // 2000, 1, 2000)
    return _pool_readout_call(h, batch3, W1, b1.reshape(1, F),
                              W2, b2.reshape(1, 1))
